# baseline (speedup 1.0000x reference)
import functools

import jax
import jax.numpy as jnp
from jax import lax
from jax.experimental import pallas as pl
from jax.experimental.pallas import tpu as pltpu
from jax.experimental.pallas import tpu_sc as plsc

VOCAB = 100000
HIDDEN = 128
BATCH = 4096
SEQ = 50
SEQ_PAD = 56
EPS = 1e-3

NUM_WORKERS = 32
ROWS_PER_WORKER = BATCH // NUM_WORKERS
LANES = 16
NCHUNK = HIDDEN // LANES


def _sc_pool_body(idx_hbm, table_hbm, out_hbm, idx_v, g0, g1, pooled, sem0, sem1):
    nc = 2
    wid = lax.axis_index("s") * nc + lax.axis_index("c")
    base = wid * ROWS_PER_WORKER

    pltpu.sync_copy(idx_hbm.at[pl.ds(base, ROWS_PER_WORKER)], idx_v)

    def gather(b, buf, sem):
        return pltpu.make_async_copy(table_hbm.at[idx_v.at[b]], buf, sem)

    def accumulate(buf, b):
        accs = [buf[0, pl.ds(c * LANES, LANES)] for c in range(NCHUNK)]
        for l in range(1, SEQ):
            for c in range(NCHUNK):
                accs[c] = accs[c] + buf[l, pl.ds(c * LANES, LANES)]
        for c in range(NCHUNK):
            pooled[b, pl.ds(c * LANES, LANES)] = accs[c] * (1.0 / SEQ)

    gather(0, g0, sem0).start()

    def loop_body(i, _):
        b = 2 * i
        gather(b, g0, sem0).wait()
        gather(b + 1, g1, sem1).start()
        accumulate(g0, b)
        gather(b + 1, g1, sem1).wait()
        nxt = jnp.minimum(b + 2, ROWS_PER_WORKER - 1)
        gather(nxt, g0, sem0).start()
        accumulate(g1, b + 1)
        return _

    lax.fori_loop(0, ROWS_PER_WORKER // 2, loop_body, None)
    gather(ROWS_PER_WORKER - 1, g0, sem0).wait()

    pltpu.sync_copy(pooled, out_hbm.at[pl.ds(base, ROWS_PER_WORKER)])


@jax.jit
def _sc_pool(idx_padded, table):
    mesh = plsc.VectorSubcoreMesh(core_axis_name="c", subcore_axis_name="s")
    return pl.kernel(
        _sc_pool_body,
        mesh=mesh,
        out_type=jax.ShapeDtypeStruct((BATCH, HIDDEN), jnp.float32),
        scratch_types=[
            pltpu.VMEM((ROWS_PER_WORKER, SEQ_PAD), jnp.int32),
            pltpu.VMEM((SEQ_PAD, HIDDEN), jnp.float32),
            pltpu.VMEM((SEQ_PAD, HIDDEN), jnp.float32),
            pltpu.VMEM((ROWS_PER_WORKER, HIDDEN), jnp.float32),
            pltpu.SemaphoreType.DMA,
            pltpu.SemaphoreType.DMA,
        ],
    )(idx_padded, table)


def _tc_layernorm_body(x_ref, gamma_ref, beta_ref, o_ref):
    x = x_ref[:, :]
    mu = jnp.mean(x, axis=0, keepdims=True)
    d = x - mu
    var = jnp.mean(d * d, axis=0, keepdims=True)
    xn = d * lax.rsqrt(var + EPS)
    o_ref[:, :] = xn * gamma_ref[:, :] + beta_ref[:, :]


@jax.jit
def _tc_layernorm(x, gamma, beta):
    return pl.pallas_call(
        _tc_layernorm_body,
        out_shape=jax.ShapeDtypeStruct((BATCH, HIDDEN), jnp.float32),
    )(x, gamma.reshape(BATCH, 1), beta.reshape(BATCH, 1))


def kernel(inputs, table, bias, gamma, beta):
    del bias
    idx_padded = jnp.concatenate(
        [inputs, jnp.zeros((BATCH, SEQ_PAD - SEQ), jnp.int32)], axis=1
    )
    pooled = _sc_pool(idx_padded, table)
    return _tc_layernorm(pooled, gamma, beta)
